# baseline (speedup 1.0000x reference)
import functools

import jax
import jax.numpy as jnp
from jax import lax
from jax.experimental import pallas as pl
from jax.experimental.pallas import tpu as pltpu
from jax.experimental.pallas import tpu_sc as plsc

N = 10000
D = 128
E = 320000
BN_EPS = 1e-5

NC = 2
NS = 16
DH = D // NC
EPS_ = E // NS
CH = 80
NCHUNK = EPS_ // CH
DEPTH = 5
N_PAD = 10240
RPS = N_PAD // NS


def _sc_aggregate(xs, src, dst3, zrows):
    mesh = plsc.VectorSubcoreMesh(core_axis_name="c", subcore_axis_name="s",
                                  num_cores=NC, num_subcores=NS)

    @functools.partial(
        pl.kernel,
        out_type=jax.ShapeDtypeStruct((NC, N_PAD, DH), jnp.float32),
        mesh=mesh,
        scratch_types=(
            [pltpu.VMEM((EPS_,), jnp.int32),
             pltpu.VMEM((NCHUNK, CH), jnp.int32)]
            + [pltpu.VMEM((CH, DH), jnp.float32)] * DEPTH
            + [pltpu.VMEM_SHARED((N_PAD, DH), jnp.float32)]
            + [pltpu.SemaphoreType.DMA] * DEPTH
        ),
        compiler_params=pltpu.CompilerParams(use_tc_tiling_on_sc=False),
    )
    def agg_kernel(x_hbm, src_hbm, dst_hbm, z_hbm, out_hbm,
                   src_v, dst_v, *rest):
        bufs = rest[:DEPTH]
        acc = rest[DEPTH]
        sems = rest[DEPTH + 1:]
        cid = lax.axis_index("c")
        sid = lax.axis_index("s")
        base = sid * EPS_
        xh = x_hbm.at[cid]

        pltpu.sync_copy(src_hbm.at[pl.ds(base, EPS_)], src_v)
        pltpu.sync_copy(dst_hbm.at[sid], dst_v)
        pltpu.sync_copy(z_hbm, acc.at[pl.ds(sid * RPS, RPS)])
        plsc.subcore_barrier()

        for k in range(DEPTH):
            pltpu.async_copy(xh.at[src_v.at[pl.ds(k * CH, CH)]],
                             bufs[k], sems[k])

        @pl.loop(0, NCHUNK - DEPTH, step=DEPTH)
        def _(g):
            for k in range(DEPTH):
                pltpu.make_async_copy(
                    xh.at[pl.ds(0, CH)], bufs[k], sems[k]).wait()
                pltpu.sync_copy(bufs[k], acc.at[dst_v.at[g + k]], add=True)
                pltpu.async_copy(
                    xh.at[src_v.at[pl.ds((g + DEPTH + k) * CH, CH)]],
                    bufs[k], sems[k])

        for k in range(DEPTH):
            pltpu.make_async_copy(
                xh.at[pl.ds(0, CH)], bufs[k], sems[k]).wait()
            pltpu.sync_copy(bufs[k], acc.at[dst_v.at[NCHUNK - DEPTH + k]],
                            add=True)

        plsc.subcore_barrier()
        pltpu.sync_copy(acc.at[pl.ds(sid * RPS, RPS)],
                        out_hbm.at[cid].at[pl.ds(sid * RPS, RPS)])

    return agg_kernel(xs, src, dst3, zrows)


BM = 1000


def _mlp(xs, a, w, b):
    def body(x_ref, a_ref, w_ref, b_ref, o_ref):
        h = jnp.concatenate([x_ref[0] + a_ref[0], x_ref[1] + a_ref[1]], axis=1)
        y = jnp.dot(h, w_ref[...], preferred_element_type=jnp.float32)
        y = jnp.maximum(y + b_ref[...], 0.0)
        o_ref[0] = y[:, :DH]
        o_ref[1] = y[:, DH:]

    return pl.pallas_call(
        body,
        grid=(N // BM,),
        in_specs=[
            pl.BlockSpec((NC, BM, DH), lambda i: (0, i, 0)),
            pl.BlockSpec((NC, BM, DH), lambda i: (0, i, 0)),
            pl.BlockSpec((D, D), lambda i: (0, 0)),
            pl.BlockSpec((1, D), lambda i: (0, 0)),
        ],
        out_specs=pl.BlockSpec((NC, BM, DH), lambda i: (0, i, 0)),
        out_shape=jax.ShapeDtypeStruct((NC, N, DH), jnp.float32),
    )(xs, a, w, b)


def _tail(xs, a, w, b, wl1, bl1, wl2, bl2):
    def body(x_ref, a_ref, w_ref, b_ref,
             wl1_ref, bl1_ref, wl2_ref, bl2_ref, o_ref):
        h = jnp.concatenate([x_ref[0] + a_ref[0], x_ref[1] + a_ref[1]], axis=1)
        t = jnp.dot(h, w_ref[...], preferred_element_type=jnp.float32)
        t = jnp.maximum(t + b_ref[...], 0.0)
        t = jnp.dot(t, wl1_ref[...], preferred_element_type=jnp.float32)
        t = jnp.maximum(t + bl1_ref[...], 0.0)
        t = jnp.dot(t, wl2_ref[...], preferred_element_type=jnp.float32)
        o_ref[...] = t + bl2_ref[...]

    full = lambda i: (0, 0)
    return pl.pallas_call(
        body,
        grid=(N // BM,),
        in_specs=[
            pl.BlockSpec((NC, BM, DH), lambda i: (0, i, 0)),
            pl.BlockSpec((NC, BM, DH), lambda i: (0, i, 0)),
            pl.BlockSpec((D, D), full),
            pl.BlockSpec((1, D), full),
            pl.BlockSpec((D, D), full),
            pl.BlockSpec((1, D), full),
            pl.BlockSpec((D, D), full),
            pl.BlockSpec((1, D), full),
        ],
        out_specs=pl.BlockSpec((BM, D), lambda i: (i, 0)),
        out_shape=jax.ShapeDtypeStruct((N, D), jnp.float32),
    )(xs, a, w, b, wl1, bl1, wl2, bl2)


def _fold_bn(w, b, g, bt, m, v):
    s = g / jnp.sqrt(v + BN_EPS)
    return w * s[None, :], ((b - m) * s + bt)[None, :]


def kernel(x, edge_index,
           W0, b0, g0, bt0, m0, v0,
           W1, b1, g1, bt1, m1, v1,
           W2, b2, g2, bt2, m2, v2,
           Wl1, bl1, Wl2, bl2):
    src = edge_index[0]
    dst3 = edge_index[1].reshape(NS, NCHUNK, CH)
    zrows = jnp.zeros((RPS, DH), dtype=jnp.float32)

    w0, c0 = _fold_bn(W0, b0, g0, bt0, m0, v0)
    w1, c1 = _fold_bn(W1, b1, g1, bt1, m1, v1)
    w2, c2 = _fold_bn(W2, b2, g2, bt2, m2, v2)

    xs = jnp.stack([x[:, :DH], x[:, DH:]])
    a = _sc_aggregate(xs, src, dst3, zrows)
    xs = _mlp(xs, a, w0, c0)
    a = _sc_aggregate(xs, src, dst3, zrows)
    xs = _mlp(xs, a, w1, c1)
    a = _sc_aggregate(xs, src, dst3, zrows)
    return _tail(xs, a, w2, c2, Wl1, bl1[None, :], Wl2, bl2[None, :])


# speedup vs baseline: 1.1520x; 1.1520x over previous
import functools

import jax
import jax.numpy as jnp
from jax import lax
from jax.experimental import pallas as pl
from jax.experimental.pallas import tpu as pltpu
from jax.experimental.pallas import tpu_sc as plsc

N = 10000
D = 128
E = 320000
BN_EPS = 1e-5

NC = 2
NS = 16
EPS_ = E // (NC * NS)
CH = 40
NCHUNK = EPS_ // CH
DEPTH = 5
N_PAD = 10240
RPS = N_PAD // NS


def _sc_aggregate(x, src2, dst4, zrows):
    mesh = plsc.VectorSubcoreMesh(core_axis_name="c", subcore_axis_name="s",
                                  num_cores=NC, num_subcores=NS)

    @functools.partial(
        pl.kernel,
        out_type=jax.ShapeDtypeStruct((NC, N_PAD, D), jnp.float32),
        mesh=mesh,
        scratch_types=(
            [pltpu.VMEM((EPS_,), jnp.int32),
             pltpu.VMEM((NCHUNK, CH), jnp.int32)]
            + [pltpu.VMEM((CH, D), jnp.float32)] * DEPTH
            + [pltpu.VMEM_SHARED((N_PAD, D), jnp.float32)]
            + [pltpu.SemaphoreType.DMA] * DEPTH
        ),
        compiler_params=pltpu.CompilerParams(use_tc_tiling_on_sc=False),
    )
    def agg_kernel(x_hbm, src_hbm, dst_hbm, z_hbm, out_hbm,
                   src_v, dst_v, *rest):
        bufs = rest[:DEPTH]
        acc = rest[DEPTH]
        sems = rest[DEPTH + 1:]
        cid = lax.axis_index("c")
        sid = lax.axis_index("s")

        pltpu.sync_copy(src_hbm.at[cid].at[sid], src_v)
        pltpu.sync_copy(dst_hbm.at[cid].at[sid], dst_v)
        pltpu.sync_copy(z_hbm, acc.at[pl.ds(sid * RPS, RPS)])
        plsc.subcore_barrier()

        for k in range(DEPTH):
            pltpu.async_copy(x_hbm.at[src_v.at[pl.ds(k * CH, CH)]],
                             bufs[k], sems[k])

        @pl.loop(0, NCHUNK - DEPTH, step=DEPTH)
        def _(g):
            for k in range(DEPTH):
                pltpu.make_async_copy(
                    x_hbm.at[pl.ds(0, CH)], bufs[k], sems[k]).wait()
                pltpu.sync_copy(bufs[k], acc.at[dst_v.at[g + k]], add=True)
                pltpu.async_copy(
                    x_hbm.at[src_v.at[pl.ds((g + DEPTH + k) * CH, CH)]],
                    bufs[k], sems[k])

        for k in range(DEPTH):
            pltpu.make_async_copy(
                x_hbm.at[pl.ds(0, CH)], bufs[k], sems[k]).wait()
            pltpu.sync_copy(bufs[k], acc.at[dst_v.at[NCHUNK - DEPTH + k]],
                            add=True)

        plsc.subcore_barrier()
        pltpu.sync_copy(acc.at[pl.ds(sid * RPS, RPS)],
                        out_hbm.at[cid].at[pl.ds(sid * RPS, RPS)])

    return agg_kernel(x, src2, dst4, zrows)


BM = 1000


def _mlp(x, a, w, b):
    def body(x_ref, a_ref, w_ref, b_ref, o_ref):
        h = x_ref[...] + a_ref[0] + a_ref[1]
        y = jnp.dot(h, w_ref[...], preferred_element_type=jnp.float32)
        o_ref[...] = jnp.maximum(y + b_ref[...], 0.0)

    return pl.pallas_call(
        body,
        grid=(N // BM,),
        in_specs=[
            pl.BlockSpec((BM, D), lambda i: (i, 0)),
            pl.BlockSpec((NC, BM, D), lambda i: (0, i, 0)),
            pl.BlockSpec((D, D), lambda i: (0, 0)),
            pl.BlockSpec((1, D), lambda i: (0, 0)),
        ],
        out_specs=pl.BlockSpec((BM, D), lambda i: (i, 0)),
        out_shape=jax.ShapeDtypeStruct((N, D), jnp.float32),
    )(x, a, w, b)


def _tail(x, a, w, b, wl1, bl1, wl2, bl2):
    def body(x_ref, a_ref, w_ref, b_ref,
             wl1_ref, bl1_ref, wl2_ref, bl2_ref, o_ref):
        h = x_ref[...] + a_ref[0] + a_ref[1]
        t = jnp.dot(h, w_ref[...], preferred_element_type=jnp.float32)
        t = jnp.maximum(t + b_ref[...], 0.0)
        t = jnp.dot(t, wl1_ref[...], preferred_element_type=jnp.float32)
        t = jnp.maximum(t + bl1_ref[...], 0.0)
        t = jnp.dot(t, wl2_ref[...], preferred_element_type=jnp.float32)
        o_ref[...] = t + bl2_ref[...]

    full = lambda i: (0, 0)
    return pl.pallas_call(
        body,
        grid=(N // BM,),
        in_specs=[
            pl.BlockSpec((BM, D), lambda i: (i, 0)),
            pl.BlockSpec((NC, BM, D), lambda i: (0, i, 0)),
            pl.BlockSpec((D, D), full),
            pl.BlockSpec((1, D), full),
            pl.BlockSpec((D, D), full),
            pl.BlockSpec((1, D), full),
            pl.BlockSpec((D, D), full),
            pl.BlockSpec((1, D), full),
        ],
        out_specs=pl.BlockSpec((BM, D), lambda i: (i, 0)),
        out_shape=jax.ShapeDtypeStruct((N, D), jnp.float32),
    )(x, a, w, b, wl1, bl1, wl2, bl2)


def _fold_bn(w, b, g, bt, m, v):
    s = g / jnp.sqrt(v + BN_EPS)
    return w * s[None, :], ((b - m) * s + bt)[None, :]


def kernel(x, edge_index,
           W0, b0, g0, bt0, m0, v0,
           W1, b1, g1, bt1, m1, v1,
           W2, b2, g2, bt2, m2, v2,
           Wl1, bl1, Wl2, bl2):
    src2 = edge_index[0].reshape(NC, NS, EPS_)
    dst4 = edge_index[1].reshape(NC, NS, NCHUNK, CH)
    zrows = jnp.zeros((RPS, D), dtype=jnp.float32)

    w0, c0 = _fold_bn(W0, b0, g0, bt0, m0, v0)
    w1, c1 = _fold_bn(W1, b1, g1, bt1, m1, v1)
    w2, c2 = _fold_bn(W2, b2, g2, bt2, m2, v2)

    a = _sc_aggregate(x, src2, dst4, zrows)
    h = _mlp(x, a, w0, c0)
    a = _sc_aggregate(h, src2, dst4, zrows)
    h = _mlp(h, a, w1, c1)
    a = _sc_aggregate(h, src2, dst4, zrows)
    return _tail(h, a, w2, c2, Wl1, bl1[None, :], Wl2, bl2[None, :])


# speedup vs baseline: 1.1845x; 1.0282x over previous
import functools

import jax
import jax.numpy as jnp
from jax import lax
from jax.experimental import pallas as pl
from jax.experimental.pallas import tpu as pltpu
from jax.experimental.pallas import tpu_sc as plsc

N = 10000
D = 128
E = 320000
BN_EPS = 1e-5

NC = 2
NS = 16
EPS_ = E // (NC * NS)
CH = 40
NCHUNK = EPS_ // CH
DEPTH = 5
N_PAD = 10240
RPS = N_PAD // NS


def _sc_aggregate(x, src2, dst4, zrows):
    mesh = plsc.VectorSubcoreMesh(core_axis_name="c", subcore_axis_name="s",
                                  num_cores=NC, num_subcores=NS)

    @functools.partial(
        pl.kernel,
        out_type=jax.ShapeDtypeStruct((NC, N_PAD, D), jnp.float32),
        mesh=mesh,
        scratch_types=(
            [pltpu.VMEM((EPS_,), jnp.int32),
             pltpu.VMEM((NCHUNK, CH), jnp.int32)]
            + [pltpu.VMEM((CH, D), jnp.float32)] * DEPTH
            + [pltpu.VMEM_SHARED((N_PAD, D), jnp.float32)]
            + [pltpu.SemaphoreType.DMA] * DEPTH
        ),
        compiler_params=pltpu.CompilerParams(use_tc_tiling_on_sc=False),
    )
    def agg_kernel(x_hbm, src_hbm, dst_hbm, z_hbm, out_hbm,
                   src_v, dst_v, *rest):
        bufs = rest[:DEPTH]
        acc = rest[DEPTH]
        sems = rest[DEPTH + 1:]
        cid = lax.axis_index("c")
        sid = lax.axis_index("s")

        pltpu.sync_copy(src_hbm.at[cid].at[sid], src_v)
        pltpu.sync_copy(dst_hbm.at[cid].at[sid], dst_v)
        pltpu.sync_copy(z_hbm, acc.at[pl.ds(sid * RPS, RPS)])
        plsc.subcore_barrier()

        for k in range(DEPTH):
            pltpu.async_copy(x_hbm.at[src_v.at[pl.ds(k * CH, CH)]],
                             bufs[k], sems[k])

        @pl.loop(0, NCHUNK - DEPTH, step=DEPTH)
        def _(g):
            for k in range(DEPTH):
                pltpu.make_async_copy(
                    x_hbm.at[pl.ds(0, CH)], bufs[k], sems[k]).wait()
                pltpu.sync_copy(bufs[k], acc.at[dst_v.at[g + k]], add=True)
                pltpu.async_copy(
                    x_hbm.at[src_v.at[pl.ds((g + DEPTH + k) * CH, CH)]],
                    bufs[k], sems[k])

        for k in range(DEPTH):
            pltpu.make_async_copy(
                x_hbm.at[pl.ds(0, CH)], bufs[k], sems[k]).wait()
            pltpu.sync_copy(bufs[k], acc.at[dst_v.at[NCHUNK - DEPTH + k]],
                            add=True)

        plsc.subcore_barrier()
        pltpu.sync_copy(acc.at[pl.ds(sid * RPS, RPS)],
                        out_hbm.at[cid].at[pl.ds(sid * RPS, RPS)])

    return agg_kernel(x, src2, dst4, zrows)


BM = 1000
KB = 250


def _split_edges(edge_index):
    def body(e_ref, s_ref, d_ref):
        s_ref[...] = e_ref[0].reshape(E // 128, 128)
        d_ref[...] = e_ref[1].reshape(E // 128, 128)

    return pl.pallas_call(
        body,
        out_shape=[jax.ShapeDtypeStruct((E // 128, 128), jnp.int32)] * 2,
    )(edge_index)


def _mlp(x, a, w, b):
    def body(x_ref, a_ref, w_ref, b_ref, o_ref):
        h = x_ref[...] + a_ref[0] + a_ref[1]
        y = jnp.dot(h, w_ref[...], preferred_element_type=jnp.float32)
        o_ref[...] = jnp.maximum(y + b_ref[...], 0.0)

    return pl.pallas_call(
        body,
        grid=(N // BM,),
        in_specs=[
            pl.BlockSpec((BM, D), lambda i: (i, 0)),
            pl.BlockSpec((NC, BM, D), lambda i: (0, i, 0)),
            pl.BlockSpec((D, D), lambda i: (0, 0)),
            pl.BlockSpec((1, D), lambda i: (0, 0)),
        ],
        out_specs=pl.BlockSpec((BM, D), lambda i: (i, 0)),
        out_shape=jax.ShapeDtypeStruct((N, D), jnp.float32),
    )(x, a, w, b)


def _tail(x, a, w, b, wl1, bl1, wl2, bl2):
    def body(x_ref, a_ref, w_ref, b_ref,
             wl1_ref, bl1_ref, wl2_ref, bl2_ref, o_ref):
        h = x_ref[...] + a_ref[0] + a_ref[1]
        t = jnp.dot(h, w_ref[...], preferred_element_type=jnp.float32)
        t = jnp.maximum(t + b_ref[...], 0.0)
        t = jnp.dot(t, wl1_ref[...], preferred_element_type=jnp.float32)
        t = jnp.maximum(t + bl1_ref[...], 0.0)
        t = jnp.dot(t, wl2_ref[...], preferred_element_type=jnp.float32)
        o_ref[...] = t + bl2_ref[...]

    full = lambda i: (0, 0)
    return pl.pallas_call(
        body,
        grid=(N // BM,),
        in_specs=[
            pl.BlockSpec((BM, D), lambda i: (i, 0)),
            pl.BlockSpec((NC, BM, D), lambda i: (0, i, 0)),
            pl.BlockSpec((D, D), full),
            pl.BlockSpec((1, D), full),
            pl.BlockSpec((D, D), full),
            pl.BlockSpec((1, D), full),
            pl.BlockSpec((D, D), full),
            pl.BlockSpec((1, D), full),
        ],
        out_specs=pl.BlockSpec((BM, D), lambda i: (i, 0)),
        out_shape=jax.ShapeDtypeStruct((N, D), jnp.float32),
    )(x, a, w, b, wl1, bl1, wl2, bl2)


def _fold_bn(w, b, g, bt, m, v):
    s = g / jnp.sqrt(v + BN_EPS)
    return w * s[None, :], ((b - m) * s + bt)[None, :]


def kernel(x, edge_index,
           W0, b0, g0, bt0, m0, v0,
           W1, b1, g1, bt1, m1, v1,
           W2, b2, g2, bt2, m2, v2,
           Wl1, bl1, Wl2, bl2):
    src_p, dst_p = _split_edges(edge_index)
    src2 = src_p.reshape(NC, NS, EPS_)
    dst4 = dst_p.reshape(NC, NS, NCHUNK, CH)
    zrows = jnp.zeros((RPS, D), dtype=jnp.float32)

    w0, c0 = _fold_bn(W0, b0, g0, bt0, m0, v0)
    w1, c1 = _fold_bn(W1, b1, g1, bt1, m1, v1)
    w2, c2 = _fold_bn(W2, b2, g2, bt2, m2, v2)

    a = _sc_aggregate(x, src2, dst4, zrows)
    h = _mlp(x, a, w0, c0)
    a = _sc_aggregate(h, src2, dst4, zrows)
    h = _mlp(h, a, w1, c1)
    a = _sc_aggregate(h, src2, dst4, zrows)
    return _tail(h, a, w2, c2, Wl1, bl1[None, :], Wl2, bl2[None, :])


# speedup vs baseline: 1.1950x; 1.0089x over previous
import functools

import jax
import jax.numpy as jnp
from jax import lax
from jax.experimental import pallas as pl
from jax.experimental.pallas import tpu as pltpu
from jax.experimental.pallas import tpu_sc as plsc

N = 10000
D = 128
E = 320000
BN_EPS = 1e-5

NC = 2
NS = 16
EPS_ = E // (NC * NS)
CH = 40
NCHUNK = EPS_ // CH
DEPTH = 5
N_PAD = 10240
RPS = N_PAD // NS


def _sc_aggregate(x, src2, dst4, zrows):
    mesh = plsc.VectorSubcoreMesh(core_axis_name="c", subcore_axis_name="s",
                                  num_cores=NC, num_subcores=NS)

    @functools.partial(
        pl.kernel,
        out_type=jax.ShapeDtypeStruct((NC, N_PAD, D), jnp.float32),
        mesh=mesh,
        scratch_types=(
            [pltpu.VMEM((EPS_,), jnp.int32),
             pltpu.VMEM((NCHUNK, CH), jnp.int32)]
            + [pltpu.VMEM((CH, D), jnp.float32)] * DEPTH
            + [pltpu.VMEM_SHARED((N_PAD, D), jnp.float32)]
            + [pltpu.SemaphoreType.DMA] * DEPTH
        ),
        compiler_params=pltpu.CompilerParams(use_tc_tiling_on_sc=False),
    )
    def agg_kernel(x_hbm, src_hbm, dst_hbm, z_hbm, out_hbm,
                   src_v, dst_v, *rest):
        bufs = rest[:DEPTH]
        acc = rest[DEPTH]
        sems = rest[DEPTH + 1:]
        cid = lax.axis_index("c")
        sid = lax.axis_index("s")

        pltpu.sync_copy(src_hbm.at[cid].at[sid], src_v)
        pltpu.sync_copy(dst_hbm.at[cid].at[sid], dst_v)

        for k in range(DEPTH):
            pltpu.async_copy(x_hbm.at[src_v.at[pl.ds(k * CH, CH)]],
                             bufs[k], sems[k])

        pltpu.sync_copy(z_hbm, acc.at[pl.ds(sid * RPS, RPS)])
        plsc.subcore_barrier()

        @pl.loop(0, NCHUNK - DEPTH, step=DEPTH)
        def _(g):
            for k in range(DEPTH):
                pltpu.make_async_copy(
                    x_hbm.at[pl.ds(0, CH)], bufs[k], sems[k]).wait()
                pltpu.sync_copy(bufs[k], acc.at[dst_v.at[g + k]], add=True)
                pltpu.async_copy(
                    x_hbm.at[src_v.at[pl.ds((g + DEPTH + k) * CH, CH)]],
                    bufs[k], sems[k])

        for k in range(DEPTH):
            pltpu.make_async_copy(
                x_hbm.at[pl.ds(0, CH)], bufs[k], sems[k]).wait()
            pltpu.sync_copy(bufs[k], acc.at[dst_v.at[NCHUNK - DEPTH + k]],
                            add=True)

        plsc.subcore_barrier()
        pltpu.sync_copy(acc.at[pl.ds(sid * RPS, RPS)],
                        out_hbm.at[cid].at[pl.ds(sid * RPS, RPS)])

    return agg_kernel(x, src2, dst4, zrows)


BM = 1000
KB = 250


def _split_edges(edge_index):
    def body(e_ref, s_ref, d_ref):
        s_ref[...] = e_ref[0].reshape(E // 128, 128)
        d_ref[...] = e_ref[1].reshape(E // 128, 128)

    return pl.pallas_call(
        body,
        out_shape=[jax.ShapeDtypeStruct((E // 128, 128), jnp.int32)] * 2,
    )(edge_index)


def _mlp(x, a, w, b):
    def body(x_ref, a_ref, w_ref, b_ref, o_ref):
        h = x_ref[...] + a_ref[0] + a_ref[1]
        y = jnp.dot(h, w_ref[...], preferred_element_type=jnp.float32)
        o_ref[...] = jnp.maximum(y + b_ref[...], 0.0)

    return pl.pallas_call(
        body,
        grid=(N // BM,),
        in_specs=[
            pl.BlockSpec((BM, D), lambda i: (i, 0)),
            pl.BlockSpec((NC, BM, D), lambda i: (0, i, 0)),
            pl.BlockSpec((D, D), lambda i: (0, 0)),
            pl.BlockSpec((1, D), lambda i: (0, 0)),
        ],
        out_specs=pl.BlockSpec((BM, D), lambda i: (i, 0)),
        out_shape=jax.ShapeDtypeStruct((N, D), jnp.float32),
    )(x, a, w, b)


def _tail(x, a, w, b, wl1, bl1, wl2, bl2):
    def body(x_ref, a_ref, w_ref, b_ref,
             wl1_ref, bl1_ref, wl2_ref, bl2_ref, o_ref):
        h = x_ref[...] + a_ref[0] + a_ref[1]
        t = jnp.dot(h, w_ref[...], preferred_element_type=jnp.float32)
        t = jnp.maximum(t + b_ref[...], 0.0)
        t = jnp.dot(t, wl1_ref[...], preferred_element_type=jnp.float32)
        t = jnp.maximum(t + bl1_ref[...], 0.0)
        t = jnp.dot(t, wl2_ref[...], preferred_element_type=jnp.float32)
        o_ref[...] = t + bl2_ref[...]

    full = lambda i: (0, 0)
    return pl.pallas_call(
        body,
        grid=(N // BM,),
        in_specs=[
            pl.BlockSpec((BM, D), lambda i: (i, 0)),
            pl.BlockSpec((NC, BM, D), lambda i: (0, i, 0)),
            pl.BlockSpec((D, D), full),
            pl.BlockSpec((1, D), full),
            pl.BlockSpec((D, D), full),
            pl.BlockSpec((1, D), full),
            pl.BlockSpec((D, D), full),
            pl.BlockSpec((1, D), full),
        ],
        out_specs=pl.BlockSpec((BM, D), lambda i: (i, 0)),
        out_shape=jax.ShapeDtypeStruct((N, D), jnp.float32),
    )(x, a, w, b, wl1, bl1, wl2, bl2)


def _fold_bn(w, b, g, bt, m, v):
    s = g / jnp.sqrt(v + BN_EPS)
    return w * s[None, :], ((b - m) * s + bt)[None, :]


def kernel(x, edge_index,
           W0, b0, g0, bt0, m0, v0,
           W1, b1, g1, bt1, m1, v1,
           W2, b2, g2, bt2, m2, v2,
           Wl1, bl1, Wl2, bl2):
    src_p, dst_p = _split_edges(edge_index)
    src2 = src_p.reshape(NC, NS, EPS_)
    dst4 = dst_p.reshape(NC, NS, NCHUNK, CH)
    zrows = jnp.zeros((RPS, D), dtype=jnp.float32)

    w0, c0 = _fold_bn(W0, b0, g0, bt0, m0, v0)
    w1, c1 = _fold_bn(W1, b1, g1, bt1, m1, v1)
    w2, c2 = _fold_bn(W2, b2, g2, bt2, m2, v2)

    a = _sc_aggregate(x, src2, dst4, zrows)
    h = _mlp(x, a, w0, c0)
    a = _sc_aggregate(h, src2, dst4, zrows)
    h = _mlp(h, a, w1, c1)
    a = _sc_aggregate(h, src2, dst4, zrows)
    return _tail(h, a, w2, c2, Wl1, bl1[None, :], Wl2, bl2[None, :])
